# baseline (speedup 1.0000x reference)
import jax
import jax.numpy as jnp
from jax import lax
from jax.experimental import pallas as pl
from jax.experimental.pallas import tpu as pltpu
from jax.experimental.pallas import tpu_sc as plsc

BATCH = 16384
D = 64
NC = 2
NS = 16
NW = NC * NS
BPW = BATCH // NW
CHUNK = 128
NCHUNK = BPW // CHUNK


def _gather_body(p_idx_hbm, t_idx_hbm, p_tab, t_tab, p_out, t_out,
                 pidx_v, tidx_v, prow_v, trow_v, sem_p, sem_t):
    wid = lax.axis_index("s") * NC + lax.axis_index("c")
    base = wid * BPW
    for j in range(NCHUNK):
        src = pl.ds(base + j * CHUNK, CHUNK)
        pltpu.sync_copy(p_idx_hbm.at[src], pidx_v.at[j])
        pltpu.sync_copy(t_idx_hbm.at[src], tidx_v.at[j])
    p_cap = jnp.full((16,), p_tab.shape[0] - 1, jnp.int32)
    t_cap = jnp.full((16,), t_tab.shape[0] - 1, jnp.int32)
    for j in range(NCHUNK):
        for i in range(CHUNK // 16):
            s = pl.ds(i * 16, 16)
            pidx_v[j, s] = jnp.minimum(pidx_v[j, s], p_cap)
            tidx_v[j, s] = jnp.minimum(tidx_v[j, s], t_cap)
    copies = []
    for j in range(NCHUNK):
        dst = pl.ds(j * CHUNK, CHUNK)
        copies.append(pltpu.async_copy(p_tab.at[pidx_v.at[j]],
                                       prow_v.at[dst], sem_p))
        copies.append(pltpu.async_copy(t_tab.at[tidx_v.at[j]],
                                       trow_v.at[dst], sem_t))
    for cp in copies:
        cp.wait()
    pltpu.sync_copy(prow_v, p_out.at[pl.ds(base, BPW)])
    pltpu.sync_copy(trow_v, t_out.at[pl.ds(base, BPW)])


def _sc_gather(p_idx, t_idx, p_tab, t_tab):
    mesh = plsc.VectorSubcoreMesh(core_axis_name="c", subcore_axis_name="s")
    k = pl.kernel(
        _gather_body,
        out_type=[jax.ShapeDtypeStruct((BATCH, D), jnp.float32),
                  jax.ShapeDtypeStruct((BATCH, D), jnp.float32)],
        mesh=mesh,
        scratch_types=[
            pltpu.VMEM((NCHUNK, CHUNK), jnp.int32),
            pltpu.VMEM((NCHUNK, CHUNK), jnp.int32),
            pltpu.VMEM((BPW, D), jnp.float32),
            pltpu.VMEM((BPW, D), jnp.float32),
            pltpu.SemaphoreType.DMA,
            pltpu.SemaphoreType.DMA,
        ],
        compiler_params=pltpu.CompilerParams(use_tc_tiling_on_sc=False),
    )
    return k(p_idx, t_idx, p_tab, t_tab)


def _mlp_body(p_ref, t_ref, wp_ref, wt_ref, bp_ref, bt_ref, o_ref):
    pv = lax.dot_general(p_ref[...], wp_ref[...], (((1,), (1,)), ((), ())),
                         preferred_element_type=jnp.float32)
    tv = lax.dot_general(t_ref[...], wt_ref[...], (((1,), (1,)), ((), ())),
                         preferred_element_type=jnp.float32)
    pv = jnp.maximum(pv + bp_ref[...], 0.0)
    tv = jnp.maximum(tv + bt_ref[...], 0.0)
    o_ref[...] = jnp.sum(pv * tv, axis=1).reshape(o_ref.shape)


def _tc_mlp_dot(p_emb, t_emb, Wp, bp, Wt, bt):
    nblk = 16
    blk = BATCH // nblk
    rows = blk // 128
    out = pl.pallas_call(
        _mlp_body,
        grid=(nblk,),
        in_specs=[
            pl.BlockSpec((blk, D), lambda i: (i, 0)),
            pl.BlockSpec((blk, D), lambda i: (i, 0)),
            pl.BlockSpec((D, D), lambda i: (0, 0)),
            pl.BlockSpec((D, D), lambda i: (0, 0)),
            pl.BlockSpec((1, D), lambda i: (0, 0)),
            pl.BlockSpec((1, D), lambda i: (0, 0)),
        ],
        out_specs=pl.BlockSpec((rows, 128), lambda i: (i, 0)),
        out_shape=jax.ShapeDtypeStruct((BATCH // 128, 128), jnp.float32),
    )(p_emb, t_emb, Wp, Wt, bp.reshape(1, D), bt.reshape(1, D))
    return out.reshape(BATCH)


def kernel(p_idx, t_idx, play_table, track_table, Wp, bp, Wt, bt):
    p_emb, t_emb = _sc_gather(p_idx, t_idx, play_table, track_table)
    return _tc_mlp_dot(p_emb, t_emb, Wp, bp, Wt, bt)


# speedup vs baseline: 1.4470x; 1.4470x over previous
import jax
import jax.numpy as jnp
from jax import lax
from jax.experimental import pallas as pl
from jax.experimental.pallas import tpu as pltpu
from jax.experimental.pallas import tpu_sc as plsc

BATCH = 16384
D = 64
NC = 2
NS = 16
NW = NC * NS
BPW = BATCH // NW
K = 16
G = BPW // K


def _gather_one(idx_v, tab, out, base, stage, buf, sem0, sem1):
    cap = tab.shape[0] - 1

    def fire(g, sem, off):
        vi = pl.multiple_of(g * K, K)
        v = jnp.minimum(idx_v[pl.ds(vi, K)], cap)
        for j in range(K):
            q = pl.multiple_of((v[j] >> 3) * 8, 8)
            pltpu.async_copy(tab.at[pl.ds(q, 8)],
                             buf.at[pl.ds(off + j * 8, 8)], sem)

    def drain(sem, off):
        pltpu.make_async_copy(tab.at[pl.ds(0, K * 8)],
                              buf.at[pl.ds(off, K * 8)], sem).wait()

    def extract(g, off):
        vi = pl.multiple_of(g * K, K)
        v = jnp.minimum(idx_v[pl.ds(vi, K)], cap)
        for j in range(K):
            s = off + j * 8 + (v[j] & 7)
            for h in range(D // 16):
                dst = pl.ds((g * K + j) * D + h * 16, 16)
                stage[dst] = buf[s, pl.ds(h * 16, 16)]

    def body(g, _):
        even = (g & 1) == 0

        @pl.when(jnp.logical_and(g < G, even))
        def _():
            fire(g, sem0, 0)

        @pl.when(jnp.logical_and(g < G, jnp.logical_not(even)))
        def _():
            fire(g, sem1, K * 8)

        @pl.when(jnp.logical_and(g > 0, even))
        def _():
            drain(sem1, K * 8)
            extract(g - 1, K * 8)

        @pl.when(jnp.logical_and(g > 0, jnp.logical_not(even)))
        def _():
            drain(sem0, 0)
            extract(g - 1, 0)

        return 0

    lax.fori_loop(0, G + 1, body, 0)
    pltpu.sync_copy(stage, out.at[pl.ds(base * D, BPW * D)])


def _gather_body(p_idx_hbm, t_idx_hbm, p_tab, t_tab, p_out, t_out,
                 pidx_v, tidx_v, buf, stage, sem0, sem1):
    wid = lax.axis_index("s") * NC + lax.axis_index("c")
    base = wid * BPW
    pltpu.sync_copy(p_idx_hbm.at[pl.ds(base, BPW)], pidx_v)
    pltpu.sync_copy(t_idx_hbm.at[pl.ds(base, BPW)], tidx_v)
    _gather_one(pidx_v, p_tab, p_out, base, stage, buf, sem0, sem1)
    _gather_one(tidx_v, t_tab, t_out, base, stage, buf, sem0, sem1)


def _sc_gather(p_idx, t_idx, p_tab, t_tab):
    mesh = plsc.VectorSubcoreMesh(core_axis_name="c", subcore_axis_name="s")
    k = pl.kernel(
        _gather_body,
        out_type=[jax.ShapeDtypeStruct((BATCH * D,), jnp.float32),
                  jax.ShapeDtypeStruct((BATCH * D,), jnp.float32)],
        mesh=mesh,
        scratch_types=[
            pltpu.VMEM((BPW,), jnp.int32),
            pltpu.VMEM((BPW,), jnp.int32),
            pltpu.VMEM((2 * K * 8, D), jnp.float32),
            pltpu.VMEM((BPW * D,), jnp.float32),
            pltpu.SemaphoreType.DMA,
            pltpu.SemaphoreType.DMA,
        ],
    )
    return k(p_idx, t_idx, p_tab, t_tab)


def _mlp_body(p_ref, t_ref, wp_ref, wt_ref, bp_ref, bt_ref, s2_ref, o_ref):
    pv = jnp.maximum(
        jnp.dot(p_ref[...], wp_ref[...], precision=lax.Precision.HIGHEST,
                preferred_element_type=jnp.float32) + bp_ref[...], 0.0)
    tv = jnp.maximum(
        jnp.dot(t_ref[...], wt_ref[...], precision=lax.Precision.HIGHEST,
                preferred_element_type=jnp.float32) + bt_ref[...], 0.0)
    o_ref[...] = jnp.dot(pv * tv, s2_ref[...],
                         precision=lax.Precision.HIGHEST,
                         preferred_element_type=jnp.float32)


def _tc_mlp_dot(p_emb2, t_emb2, W2p, b2p, W2t, b2t, S2):
    nrow = BATCH // 2
    nblk = 16
    blk = nrow // nblk
    out = pl.pallas_call(
        _mlp_body,
        grid=(nblk,),
        in_specs=[
            pl.BlockSpec((blk, 128), lambda i: (i, 0)),
            pl.BlockSpec((blk, 128), lambda i: (i, 0)),
            pl.BlockSpec((128, 128), lambda i: (0, 0)),
            pl.BlockSpec((128, 128), lambda i: (0, 0)),
            pl.BlockSpec((1, 128), lambda i: (0, 0)),
            pl.BlockSpec((1, 128), lambda i: (0, 0)),
            pl.BlockSpec((128, 2), lambda i: (0, 0)),
        ],
        out_specs=pl.BlockSpec((blk, 2), lambda i: (i, 0)),
        out_shape=jax.ShapeDtypeStruct((nrow, 2), jnp.float32),
    )(p_emb2, t_emb2, W2p, W2t, b2p, b2t, S2)
    return out.reshape(BATCH)


def kernel(p_idx, t_idx, play_table, track_table, Wp, bp, Wt, bt):
    p_flat, t_flat = _sc_gather(p_idx, t_idx, play_table, track_table)
    p_emb2 = p_flat.reshape(BATCH // 2, 128)
    t_emb2 = t_flat.reshape(BATCH // 2, 128)
    zeros = jnp.zeros((D, D), jnp.float32)
    W2p = jnp.block([[Wp.T, zeros], [zeros, Wp.T]])
    W2t = jnp.block([[Wt.T, zeros], [zeros, Wt.T]])
    b2p = jnp.concatenate([bp, bp]).reshape(1, 128)
    b2t = jnp.concatenate([bt, bt]).reshape(1, 128)
    ones = jnp.ones((D, 1), jnp.float32)
    zcol = jnp.zeros((D, 1), jnp.float32)
    S2 = jnp.block([[ones, zcol], [zcol, ones]])
    return _tc_mlp_dot(p_emb2, t_emb2, W2p, b2p, W2t, b2t, S2)


# speedup vs baseline: 1.4655x; 1.0128x over previous
import jax
import jax.numpy as jnp
from jax import lax
from jax.experimental import pallas as pl
from jax.experimental.pallas import tpu as pltpu
from jax.experimental.pallas import tpu_sc as plsc

BATCH = 16384
D = 64
NC = 2
NS = 16
NW = NC * NS
BPW = BATCH // NW
K = 16
G = BPW // K


def _gather_one(idx_v, tab, out, base, stage, buf, sem0, sem1):
    cap = tab.shape[0] - 1

    def fire(g, sem, off):
        vi = pl.multiple_of(g * K, K)
        v = jnp.minimum(idx_v[pl.ds(vi, K)], cap)
        for j in range(K):
            q = pl.multiple_of((v[j] >> 3) * 8, 8)
            pltpu.async_copy(tab.at[pl.ds(q, 8)],
                             buf.at[pl.ds(off + j * 8, 8)], sem)

    def drain(sem, off):
        pltpu.make_async_copy(tab.at[pl.ds(0, K * 8)],
                              buf.at[pl.ds(off, K * 8)], sem).wait()

    def extract(g, off):
        vi = pl.multiple_of(g * K, K)
        v = jnp.minimum(idx_v[pl.ds(vi, K)], cap)
        for j in range(K):
            s = off + j * 8 + (v[j] & 7)
            for h in range(D // 16):
                dst = pl.ds((g * K + j) * D + h * 16, 16)
                stage[dst] = buf[s, pl.ds(h * 16, 16)]

    def body(g, _):
        even = (g & 1) == 0

        @pl.when(jnp.logical_and(g < G, even))
        def _():
            fire(g, sem0, 0)

        @pl.when(jnp.logical_and(g < G, jnp.logical_not(even)))
        def _():
            fire(g, sem1, K * 8)

        @pl.when(jnp.logical_and(g > 0, even))
        def _():
            drain(sem1, K * 8)
            extract(g - 1, K * 8)

        @pl.when(jnp.logical_and(g > 0, jnp.logical_not(even)))
        def _():
            drain(sem0, 0)
            extract(g - 1, 0)

        return 0

    lax.fori_loop(0, G + 1, body, 0)
    pltpu.sync_copy(stage, out.at[pl.ds(base * D, BPW * D)])


def _gather_body(p_idx_hbm, t_idx_hbm, p_tab, t_tab, p_out, t_out,
                 pidx_v, tidx_v, buf, stage, sem0, sem1):
    wid = lax.axis_index("s") * NC + lax.axis_index("c")
    base = wid * BPW
    pltpu.sync_copy(p_idx_hbm.at[pl.ds(base, BPW)], pidx_v)
    pltpu.sync_copy(t_idx_hbm.at[pl.ds(base, BPW)], tidx_v)
    _gather_one(pidx_v, p_tab, p_out, base, stage, buf, sem0, sem1)
    _gather_one(tidx_v, t_tab, t_out, base, stage, buf, sem0, sem1)


def _sc_gather(p_idx, t_idx, p_tab, t_tab):
    mesh = plsc.VectorSubcoreMesh(core_axis_name="c", subcore_axis_name="s")
    k = pl.kernel(
        _gather_body,
        out_type=[jax.ShapeDtypeStruct((BATCH * D,), jnp.float32),
                  jax.ShapeDtypeStruct((BATCH * D,), jnp.float32)],
        mesh=mesh,
        scratch_types=[
            pltpu.VMEM((BPW,), jnp.int32),
            pltpu.VMEM((BPW,), jnp.int32),
            pltpu.VMEM((2 * K * 8, D), jnp.float32),
            pltpu.VMEM((BPW * D,), jnp.float32),
            pltpu.SemaphoreType.DMA,
            pltpu.SemaphoreType.DMA,
        ],
    )
    return k(p_idx, t_idx, p_tab, t_tab)


def _mlp_body(p_ref, t_ref, wp_ref, wt_ref, bp_ref, bt_ref, o_ref):
    xl, xr = p_ref[:, :D], p_ref[:, D:]
    yl, yr = t_ref[:, :D], t_ref[:, D:]
    dn = (((1,), (1,)), ((), ()))
    pvl = jnp.maximum(lax.dot_general(xl, wp_ref[...], dn,
                      preferred_element_type=jnp.float32) + bp_ref[...], 0.)
    pvr = jnp.maximum(lax.dot_general(xr, wp_ref[...], dn,
                      preferred_element_type=jnp.float32) + bp_ref[...], 0.)
    tvl = jnp.maximum(lax.dot_general(yl, wt_ref[...], dn,
                      preferred_element_type=jnp.float32) + bt_ref[...], 0.)
    tvr = jnp.maximum(lax.dot_general(yr, wt_ref[...], dn,
                      preferred_element_type=jnp.float32) + bt_ref[...], 0.)
    even = jnp.sum(pvl * tvl, axis=1, keepdims=True)
    odd = jnp.sum(pvr * tvr, axis=1, keepdims=True)
    o_ref[...] = jnp.concatenate([even, odd], axis=1)


def _tc_mlp_dot(p_emb2, t_emb2, Wp, bp, Wt, bt):
    nrow = BATCH // 2
    nblk = 16
    blk = nrow // nblk
    out = pl.pallas_call(
        _mlp_body,
        grid=(nblk,),
        in_specs=[
            pl.BlockSpec((blk, 128), lambda i: (i, 0)),
            pl.BlockSpec((blk, 128), lambda i: (i, 0)),
            pl.BlockSpec((D, D), lambda i: (0, 0)),
            pl.BlockSpec((D, D), lambda i: (0, 0)),
            pl.BlockSpec((1, D), lambda i: (0, 0)),
            pl.BlockSpec((1, D), lambda i: (0, 0)),
        ],
        out_specs=pl.BlockSpec((blk, 2), lambda i: (i, 0)),
        out_shape=jax.ShapeDtypeStruct((nrow, 2), jnp.float32),
    )(p_emb2, t_emb2, Wp, Wt, bp.reshape(1, D), bt.reshape(1, D))
    return out.reshape(BATCH)


def kernel(p_idx, t_idx, play_table, track_table, Wp, bp, Wt, bt):
    p_flat, t_flat = _sc_gather(p_idx, t_idx, play_table, track_table)
    p_emb2 = p_flat.reshape(BATCH // 2, 128)
    t_emb2 = t_flat.reshape(BATCH // 2, 128)
    return _tc_mlp_dot(p_emb2, t_emb2, Wp, bp, Wt, bt)
